# baseline (speedup 1.0000x reference)
import functools

import jax
import jax.numpy as jnp
from jax import lax
from jax.experimental import pallas as pl
from jax.experimental.pallas import tpu as pltpu
from jax.experimental.pallas import tpu_sc as plsc

_B, _S, _D = 4, 2048, 768
_H, _HD = 12, 64
_TKL, _LEN = 3, 5
_LMAX = 100
_LPAD = 112
_NEG = -3e38
_BIGI = 2**30

_SCHUNK = 256
_GS = _S // _SCHUNK


def _tc_a_body(x_ref, pk_ref, nd_ref, wt_ref, wpp_ref, sim_ref, comb_ref,
               acc_ref):
    g = pl.program_id(0)

    @pl.when(g == 0)
    def _():
        acc_ref[...] = jnp.zeros_like(acc_ref)

    acc_ref[...] += jnp.sum(x_ref[...], axis=1)

    @pl.when(g == _GS - 1)
    def _():
        x_mean = acc_ref[...] * (1.0 / _S)
        ss = jnp.sum(x_mean * x_mean, axis=1, keepdims=True)
        x_norm = x_mean * lax.rsqrt(jnp.maximum(ss, 1e-12))
        pk = pk_ref[...]
        pss = jnp.sum(pk * pk, axis=1, keepdims=True)
        pk_norm = pk * lax.rsqrt(jnp.maximum(pss, 1e-12))
        sim = lax.dot_general(x_norm, pk_norm, (((1,), (1,)), ((), ())),
                              preferred_element_type=jnp.float32)
        col = lax.broadcasted_iota(jnp.int32, sim.shape, 1)
        sim = jnp.where(col < _LMAX, sim, _NEG)
        sim_ref[...] = lax.slice(sim, (0, 0), (_B, _LMAX))
        de = lax.dot_general(nd_ref[...], wt_ref[...], (((1,), (1,)), ((), ())),
                             preferred_element_type=jnp.float32)
        dep = lax.dot_general(de, wpp_ref[...], (((1,), (1,)), ((), ())),
                              preferred_element_type=jnp.float32)
        sim_wide = jnp.concatenate(
            [sim, jnp.zeros((_B, _D - _LPAD), jnp.float32)], axis=1)
        zrows = jnp.zeros((_DPAD - _LMAX, _D), jnp.float32)
        comb_ref[...] = jnp.concatenate(
            [de, zrows, sim_wide, jnp.zeros((8 - _B, _D), jnp.float32),
             dep, zrows], axis=0)


def _tc_c_body(bp_ref, sim_ref, wv_k_ref, wp_k_ref, bp_k_ref,
               wv_v_ref, wp_v_ref, bp_v_ref, out_ref):
    bp = bp_ref[...]
    sim = sim_ref[...]
    col = lax.broadcasted_iota(jnp.int32, sim.shape, 1)
    s3 = [jnp.sum(jnp.where(col == p, sim, 0.0), axis=1, keepdims=True)
          for p in range(_TKL)]
    acc_k_list = []
    acc_v_list = []
    for h in range(_H):
        bh = bp[:, h * _HD:(h + 1) * _HD]
        acc_k = jnp.zeros((_B, _HD), jnp.float32)
        acc_v = jnp.zeros((_B, _HD), jnp.float32)
        for p in range(_TKL):
            vk = lax.dot_general(bh, wv_k_ref[h, p], (((1,), (1,)), ((), ())),
                                 preferred_element_type=jnp.float32)
            ok = lax.dot_general(vk, wp_k_ref[h, p], (((1,), (1,)), ((), ())),
                                 preferred_element_type=jnp.float32)
            ok = ok + bp_k_ref[h, p][None, :]
            vv = lax.dot_general(bh, wv_v_ref[h, p], (((1,), (1,)), ((), ())),
                                 preferred_element_type=jnp.float32)
            ov = lax.dot_general(vv, wp_v_ref[h, p], (((1,), (1,)), ((), ())),
                                 preferred_element_type=jnp.float32)
            ov = ov + bp_v_ref[h, p][None, :]
            acc_k = acc_k + s3[p] * ok
            acc_v = acc_v + s3[p] * ov
        acc_k_list.append(acc_k)
        acc_v_list.append(acc_v)
    nk_flat = jnp.concatenate(acc_k_list, axis=1)
    nv_flat = jnp.concatenate(acc_v_list, axis=1)
    out = jnp.concatenate(
        [jnp.broadcast_to(nk_flat[None, :, :], (_LEN, _B, _D)),
         jnp.broadcast_to(nv_flat[None, :, :], (_LEN, _B, _D))], axis=0)
    out_ref[...] = out


_NVREG = _LPAD // 16
_JW = _D // 128
_DPAD = 104
_DEPOFF = _DPAD + 8
_COMB = _DEPOFF + _DPAD


def _sc_route_body(comb_hbm, out_hbm, bp_hbm, sim_v, rows_v, out_v, bp_v,
                   sem):
    wid = lax.axis_index("s") * 2 + lax.axis_index("c")

    @pl.when(wid < _JW)
    def _():
        j = wid
        pltpu.sync_copy(comb_hbm.at[pl.ds(_DPAD, 8), pl.ds(0, 128)], sim_v)
        lanes = lax.iota(jnp.int32, 16)
        ids = [lanes + i * 16 for i in range(_NVREG)]
        tops = []
        for b in range(_B):
            vs = [sim_v[b, pl.ds(i * 16, 16)] for i in range(_NVREG)]
            for _ in range(_TKL):
                m = vs[0]
                for v in vs[1:]:
                    m = jnp.maximum(m, v)
                mval = jnp.max(m)
                ridx = jnp.int32(_BIGI)
                for i in range(_NVREG):
                    cand = jnp.where(vs[i] == mval, ids[i], _BIGI)
                    ridx = jnp.minimum(ridx, jnp.min(cand))
                tops.append((mval, ridx // 8 * 8, ridx % 8))
                vs = [jnp.where(ids[i] == ridx, _NEG, vs[i])
                      for i in range(_NVREG)]
        cps = []
        for k, (_, rbase, _) in enumerate(tops):
            rb = pl.multiple_of(rbase, 8)
            cps.append(pltpu.async_copy(
                comb_hbm.at[pl.ds(rb, 8), pl.ds(j * 128, 128)],
                rows_v.at[k], sem))
            rb2 = pl.multiple_of(rbase + _DEPOFF, 8)
            cps.append(pltpu.async_copy(
                comb_hbm.at[pl.ds(rb2, 8), pl.ds(j * 128, 128)],
                rows_v.at[_B * _TKL + k], sem))
        for cp in cps:
            cp.wait()
        for b in range(_B):
            accs = [jnp.zeros((16,), jnp.float32) for _ in range(8)]
            accs2 = [jnp.zeros((16,), jnp.float32) for _ in range(8)]
            for t in range(_TKL):
                k = b * _TKL + t
                mval, _, rr = tops[k]
                for c in range(8):
                    accs[c] = accs[c] + mval * rows_v[k, rr, pl.ds(c * 16, 16)]
                    accs2[c] = accs2[c] + mval * rows_v[_B * _TKL + k, rr,
                                                        pl.ds(c * 16, 16)]
            for c in range(8):
                out_v[b, pl.ds(c * 16, 16)] = accs[c]
                bp_v[b, pl.ds(c * 16, 16)] = accs2[c]
        pltpu.sync_copy(out_v, out_hbm.at[:, pl.ds(j * 128, 128)])
        pltpu.sync_copy(bp_v, bp_hbm.at[:, pl.ds(j * 128, 128)])


_sc_route = pl.kernel(
    _sc_route_body,
    out_type=[jax.ShapeDtypeStruct((_B, _D), jnp.float32),
              jax.ShapeDtypeStruct((_B, _D), jnp.float32)],
    mesh=plsc.VectorSubcoreMesh(core_axis_name="c", subcore_axis_name="s",
                                num_cores=2, num_subcores=16),
    compiler_params=pltpu.CompilerParams(needs_layout_passes=False,
                                         skip_device_barrier=True),
    scratch_types=[
        pltpu.VMEM((8, 128), jnp.float32),
        pltpu.VMEM((2 * _B * _TKL, 8, 128), jnp.float32),
        pltpu.VMEM((_B, 128), jnp.float32),
        pltpu.VMEM((_B, 128), jnp.float32),
        pltpu.SemaphoreType.DMA,
    ],
)


def kernel(x_embed, prompt_key, new_desc_embed, w_text, w_prompt_proj,
           w_qkv_k, w_proj_k, b_proj_k, w_qkv_v, w_proj_v, b_proj_v):
    similarity, comb = pl.pallas_call(
        _tc_a_body,
        grid=(_GS,),
        in_specs=[
            pl.BlockSpec((_B, _SCHUNK, _D), lambda g: (0, g, 0)),
            pl.BlockSpec((_LPAD, _D), lambda g: (0, 0)),
            pl.BlockSpec((_LMAX, _D), lambda g: (0, 0)),
            pl.BlockSpec((_D, _D), lambda g: (0, 0)),
            pl.BlockSpec((_D, _D), lambda g: (0, 0)),
        ],
        out_specs=[
            pl.BlockSpec((_B, _LMAX), lambda g: (0, 0)),
            pl.BlockSpec((_COMB, _D), lambda g: (0, 0)),
        ],
        out_shape=[
            jax.ShapeDtypeStruct((_B, _LMAX), jnp.float32),
            jax.ShapeDtypeStruct((_COMB, _D), jnp.float32),
        ],
        scratch_shapes=[pltpu.VMEM((_B, _D), jnp.float32)],
    )(x_embed, lax.slice_in_dim(prompt_key, 0, _LPAD, axis=0),
      new_desc_embed, w_text, w_prompt_proj)

    desc_out, bp = _sc_route(comb)

    out_bp = pl.pallas_call(
        _tc_c_body,
        grid=(1,),
        in_specs=[
            pl.BlockSpec((_B, _D), lambda i: (0, 0)),
            pl.BlockSpec((_B, _LMAX), lambda i: (0, 0)),
            pl.BlockSpec((_H, _TKL, _HD, _HD), lambda i: (0, 0, 0, 0)),
            pl.BlockSpec((_H, _TKL, _HD, _HD), lambda i: (0, 0, 0, 0)),
            pl.BlockSpec((_H, _TKL, _HD), lambda i: (0, 0, 0)),
            pl.BlockSpec((_H, _TKL, _HD, _HD), lambda i: (0, 0, 0, 0)),
            pl.BlockSpec((_H, _TKL, _HD, _HD), lambda i: (0, 0, 0, 0)),
            pl.BlockSpec((_H, _TKL, _HD), lambda i: (0, 0, 0)),
        ],
        out_specs=pl.BlockSpec((2 * _LEN, _B, _D), lambda i: (0, 0, 0)),
        out_shape=jax.ShapeDtypeStruct((2 * _LEN, _B, _D), jnp.float32),
    )(bp, similarity, lax.slice_in_dim(w_qkv_k, 2 * _HD, 3 * _HD, axis=2),
      w_proj_k, b_proj_k, lax.slice_in_dim(w_qkv_v, 2 * _HD, 3 * _HD, axis=2),
      w_proj_v, b_proj_v)

    return (similarity, desc_out, jnp.transpose(out_bp, (1, 0, 2)))


# speedup vs baseline: 1.0359x; 1.0359x over previous
import functools

import jax
import jax.numpy as jnp
from jax import lax
from jax.experimental import pallas as pl
from jax.experimental.pallas import tpu as pltpu
from jax.experimental.pallas import tpu_sc as plsc

_B, _S, _D = 4, 2048, 768
_H, _HD = 12, 64
_TKL, _LEN = 3, 5
_LMAX = 100
_LPAD = 112
_NEG = -3e38
_BIGI = 2**30

_SCHUNK = 256
_GS = _S // _SCHUNK


def _tc_a_body(x_ref, pk_ref, nd_ref, wt_ref, wpp_ref, sim_ref, comb_ref,
               acc_ref):
    g = pl.program_id(0)

    @pl.when(g == 0)
    def _():
        acc_ref[...] = jnp.zeros_like(acc_ref)

    acc_ref[...] += jnp.sum(x_ref[...], axis=1)

    @pl.when(g == _GS - 1)
    def _():
        x_mean = acc_ref[...] * (1.0 / _S)
        ss = jnp.sum(x_mean * x_mean, axis=1, keepdims=True)
        x_norm = x_mean * lax.rsqrt(jnp.maximum(ss, 1e-12))
        pk = pk_ref[...]
        pss = jnp.sum(pk * pk, axis=1, keepdims=True)
        pk_norm = pk * lax.rsqrt(jnp.maximum(pss, 1e-12))
        sim = lax.dot_general(x_norm, pk_norm, (((1,), (1,)), ((), ())),
                              preferred_element_type=jnp.float32)
        col = lax.broadcasted_iota(jnp.int32, sim.shape, 1)
        sim = jnp.where(col < _LMAX, sim, _NEG)
        sim_ref[...] = lax.slice(sim, (0, 0), (_B, _LMAX))
        de = lax.dot_general(nd_ref[...], wt_ref[...], (((1,), (1,)), ((), ())),
                             preferred_element_type=jnp.float32)
        dep = lax.dot_general(de, wpp_ref[...], (((1,), (1,)), ((), ())),
                              preferred_element_type=jnp.float32)
        sim_wide = jnp.concatenate(
            [sim, jnp.zeros((_B, _D - _LPAD), jnp.float32)], axis=1)
        zrows = jnp.zeros((_DPAD - _LMAX, _D), jnp.float32)
        comb_ref[...] = jnp.concatenate(
            [de, zrows, sim_wide, jnp.zeros((8 - _B, _D), jnp.float32),
             dep, zrows], axis=0)


def _tc_c_body(bp_ref, sim_ref, wv_k_ref, wp_k_ref, bp_k_ref,
               wv_v_ref, wp_v_ref, bp_v_ref, out_ref):
    bp = bp_ref[...]
    sim = sim_ref[...]
    col = lax.broadcasted_iota(jnp.int32, sim.shape, 1)
    s3 = [jnp.sum(jnp.where(col == p, sim, 0.0), axis=1, keepdims=True)
          for p in range(_TKL)]
    acc_k_list = []
    acc_v_list = []
    for h in range(_H):
        bh = bp[:, h * _HD:(h + 1) * _HD]
        acc_k = jnp.zeros((_B, _HD), jnp.float32)
        acc_v = jnp.zeros((_B, _HD), jnp.float32)
        for p in range(_TKL):
            vk = lax.dot_general(bh, wv_k_ref[h, p], (((1,), (1,)), ((), ())),
                                 preferred_element_type=jnp.float32)
            ok = lax.dot_general(vk, wp_k_ref[h, p], (((1,), (1,)), ((), ())),
                                 preferred_element_type=jnp.float32)
            ok = ok + bp_k_ref[h, p][None, :]
            vv = lax.dot_general(bh, wv_v_ref[h, p], (((1,), (1,)), ((), ())),
                                 preferred_element_type=jnp.float32)
            ov = lax.dot_general(vv, wp_v_ref[h, p], (((1,), (1,)), ((), ())),
                                 preferred_element_type=jnp.float32)
            ov = ov + bp_v_ref[h, p][None, :]
            acc_k = acc_k + s3[p] * ok
            acc_v = acc_v + s3[p] * ov
        acc_k_list.append(acc_k)
        acc_v_list.append(acc_v)
    nk_flat = jnp.concatenate(acc_k_list, axis=1)
    nv_flat = jnp.concatenate(acc_v_list, axis=1)
    out = jnp.concatenate(
        [jnp.broadcast_to(nk_flat[None, :, :], (_LEN, _B, _D)),
         jnp.broadcast_to(nv_flat[None, :, :], (_LEN, _B, _D))], axis=0)
    out_ref[...] = out


_NVREG = _LPAD // 16
_JW = _D // 128
_DPAD = 104
_DEPOFF = _DPAD + 8
_COMB = _DEPOFF + _DPAD


def _sc_route_body(comb_hbm, out_hbm, bp_hbm, sim_v, rows_v, out_v, bp_v,
                   sem):
    wid = lax.axis_index("s") * 2 + lax.axis_index("c")

    @pl.when(wid < _JW)
    def _():
        j = wid
        pltpu.sync_copy(comb_hbm.at[pl.ds(_DPAD, 8), pl.ds(0, 128)], sim_v)
        lanes = lax.iota(jnp.int32, 16)
        ids = [lanes + i * 16 for i in range(_NVREG)]
        tops = []
        for b in range(_B):
            vs = [sim_v[b, pl.ds(i * 16, 16)] for i in range(_NVREG)]
            for _ in range(_TKL):
                m = vs[0]
                for v in vs[1:]:
                    m = jnp.maximum(m, v)
                mval = jnp.max(m)
                ridx = jnp.int32(_BIGI)
                for i in range(_NVREG):
                    cand = jnp.where(vs[i] == mval, ids[i], _BIGI)
                    ridx = jnp.minimum(ridx, jnp.min(cand))
                tops.append((mval, ridx // 8 * 8, ridx % 8))
                vs = [jnp.where(ids[i] == ridx, _NEG, vs[i])
                      for i in range(_NVREG)]
        cps = []
        for k, (_, rbase, _) in enumerate(tops):
            rb = pl.multiple_of(rbase, 8)
            cps.append(pltpu.async_copy(
                comb_hbm.at[pl.ds(rb, 8), pl.ds(j * 128, 128)],
                rows_v.at[k], sem))
            rb2 = pl.multiple_of(rbase + _DEPOFF, 8)
            cps.append(pltpu.async_copy(
                comb_hbm.at[pl.ds(rb2, 8), pl.ds(j * 128, 128)],
                rows_v.at[_B * _TKL + k], sem))
        for cp in cps:
            cp.wait()
        for b in range(_B):
            accs = [jnp.zeros((16,), jnp.float32) for _ in range(8)]
            accs2 = [jnp.zeros((16,), jnp.float32) for _ in range(8)]
            for t in range(_TKL):
                k = b * _TKL + t
                mval, _, rr = tops[k]
                for c in range(8):
                    accs[c] = accs[c] + mval * rows_v[k, rr, pl.ds(c * 16, 16)]
                    accs2[c] = accs2[c] + mval * rows_v[_B * _TKL + k, rr,
                                                        pl.ds(c * 16, 16)]
            for c in range(8):
                out_v[b, pl.ds(c * 16, 16)] = accs[c]
                bp_v[b, pl.ds(c * 16, 16)] = accs2[c]
        pltpu.sync_copy(out_v, out_hbm.at[:, pl.ds(j * 128, 128)])
        pltpu.sync_copy(bp_v, bp_hbm.at[:, pl.ds(j * 128, 128)])


_sc_route = pl.kernel(
    _sc_route_body,
    out_type=[jax.ShapeDtypeStruct((_B, _D), jnp.float32),
              jax.ShapeDtypeStruct((_B, _D), jnp.float32)],
    mesh=plsc.VectorSubcoreMesh(core_axis_name="c", subcore_axis_name="s",
                                num_cores=2, num_subcores=16),
    compiler_params=pltpu.CompilerParams(needs_layout_passes=False,
                                         skip_device_barrier=True),
    scratch_types=[
        pltpu.VMEM((8, 128), jnp.float32),
        pltpu.VMEM((2 * _B * _TKL, 8, 128), jnp.float32),
        pltpu.VMEM((_B, 128), jnp.float32),
        pltpu.VMEM((_B, 128), jnp.float32),
        pltpu.SemaphoreType.DMA,
    ],
)


def kernel(x_embed, prompt_key, new_desc_embed, w_text, w_prompt_proj,
           w_qkv_k, w_proj_k, b_proj_k, w_qkv_v, w_proj_v, b_proj_v):
    similarity, comb = pl.pallas_call(
        _tc_a_body,
        grid=(_GS,),
        in_specs=[
            pl.BlockSpec((_B, _SCHUNK, _D), lambda g: (0, g, 0)),
            pl.BlockSpec((_LPAD, _D), lambda g: (0, 0)),
            pl.BlockSpec((_LMAX, _D), lambda g: (0, 0)),
            pl.BlockSpec((_D, _D), lambda g: (0, 0)),
            pl.BlockSpec((_D, _D), lambda g: (0, 0)),
        ],
        out_specs=[
            pl.BlockSpec((_B, _LMAX), lambda g: (0, 0)),
            pl.BlockSpec((_COMB, _D), lambda g: (0, 0)),
        ],
        out_shape=[
            jax.ShapeDtypeStruct((_B, _LMAX), jnp.float32),
            jax.ShapeDtypeStruct((_COMB, _D), jnp.float32),
        ],
        scratch_shapes=[pltpu.VMEM((_B, _D), jnp.float32)],
    )(x_embed, prompt_key, new_desc_embed, w_text, w_prompt_proj)

    desc_out, bp = _sc_route(comb)

    out_bp = pl.pallas_call(
        _tc_c_body,
        grid=(1,),
        in_specs=[
            pl.BlockSpec((_B, _D), lambda i: (0, 0)),
            pl.BlockSpec((_B, _LMAX), lambda i: (0, 0)),
            pl.BlockSpec((_H, _TKL, _HD, _HD), lambda i: (0, 0, 0, 0)),
            pl.BlockSpec((_H, _TKL, _HD, _HD), lambda i: (0, 0, 0, 0)),
            pl.BlockSpec((_H, _TKL, _HD), lambda i: (0, 0, 0)),
            pl.BlockSpec((_H, _TKL, _HD, _HD), lambda i: (0, 0, 0, 0)),
            pl.BlockSpec((_H, _TKL, _HD, _HD), lambda i: (0, 0, 0, 0)),
            pl.BlockSpec((_H, _TKL, _HD), lambda i: (0, 0, 0)),
        ],
        out_specs=pl.BlockSpec((2 * _LEN, _B, _D), lambda i: (0, 0, 0)),
        out_shape=jax.ShapeDtypeStruct((2 * _LEN, _B, _D), jnp.float32),
    )(bp, similarity, lax.slice_in_dim(w_qkv_k, 2 * _HD, 3 * _HD, axis=2),
      w_proj_k, b_proj_k, lax.slice_in_dim(w_qkv_v, 2 * _HD, 3 * _HD, axis=2),
      w_proj_v, b_proj_v)

    return (similarity, desc_out, jnp.transpose(out_bp, (1, 0, 2)))


# speedup vs baseline: 1.0432x; 1.0070x over previous
import functools

import jax
import jax.numpy as jnp
from jax import lax
from jax.experimental import pallas as pl
from jax.experimental.pallas import tpu as pltpu
from jax.experimental.pallas import tpu_sc as plsc

_B, _S, _D = 4, 2048, 768
_H, _HD = 12, 64
_TKL, _LEN = 3, 5
_LMAX = 100
_LPAD = 112
_NEG = -3e38
_BIGI = 2**30

_SCHUNK = 256
_GS = _S // _SCHUNK


def _tc_a_body(x_ref, pk_ref, nd_ref, wt_ref, wpp_ref, sim_ref, comb_ref,
               acc_ref):
    g = pl.program_id(0)

    @pl.when(g == 0)
    def _():
        acc_ref[...] = jnp.zeros_like(acc_ref)

    acc_ref[...] += jnp.sum(x_ref[...], axis=1)

    @pl.when(g == _GS - 1)
    def _():
        x_mean = acc_ref[...] * (1.0 / _S)
        ss = jnp.sum(x_mean * x_mean, axis=1, keepdims=True)
        x_norm = x_mean * lax.rsqrt(jnp.maximum(ss, 1e-12))
        pk = pk_ref[...]
        pss = jnp.sum(pk * pk, axis=1, keepdims=True)
        pk_norm = pk * lax.rsqrt(jnp.maximum(pss, 1e-12))
        sim = lax.dot_general(x_norm, pk_norm, (((1,), (1,)), ((), ())),
                              preferred_element_type=jnp.float32)
        col = lax.broadcasted_iota(jnp.int32, sim.shape, 1)
        sim = jnp.where(col < _LMAX, sim, _NEG)
        sim_ref[...] = lax.slice(sim, (0, 0), (_B, _LMAX))
        de = lax.dot_general(nd_ref[...], wt_ref[...], (((1,), (1,)), ((), ())),
                             preferred_element_type=jnp.float32)
        dep = lax.dot_general(de, wpp_ref[...], (((1,), (1,)), ((), ())),
                              preferred_element_type=jnp.float32)
        sim_wide = jnp.concatenate(
            [sim, jnp.zeros((_B, _D - _LPAD), jnp.float32)], axis=1)
        zrows = jnp.zeros((_DPAD - _LMAX, _D), jnp.float32)
        comb_ref[...] = jnp.concatenate(
            [de, zrows, sim_wide, jnp.zeros((8 - _B, _D), jnp.float32),
             dep, zrows], axis=0)


def _tc_c_body(bp_ref, sim_ref, wv_k_ref, wp_k_ref, bp_k_ref,
               wv_v_ref, wp_v_ref, bp_v_ref, out_ref):
    bp = bp_ref[...]
    sim = sim_ref[...]
    col = lax.broadcasted_iota(jnp.int32, sim.shape, 1)
    s3 = [jnp.sum(jnp.where(col == p, sim, 0.0), axis=1, keepdims=True)
          for p in range(_TKL)]
    acc_k_list = []
    acc_v_list = []
    for h in range(_H):
        bh = bp[:, h * _HD:(h + 1) * _HD]
        acc_k = jnp.zeros((_B, _HD), jnp.float32)
        acc_v = jnp.zeros((_B, _HD), jnp.float32)
        for p in range(_TKL):
            vk = lax.dot_general(bh, wv_k_ref[h, p], (((1,), (1,)), ((), ())),
                                 preferred_element_type=jnp.float32)
            ok = lax.dot_general(vk, wp_k_ref[h, p], (((1,), (1,)), ((), ())),
                                 preferred_element_type=jnp.float32)
            ok = ok + bp_k_ref[h, p][None, :]
            vv = lax.dot_general(bh, wv_v_ref[h, p], (((1,), (1,)), ((), ())),
                                 preferred_element_type=jnp.float32)
            ov = lax.dot_general(vv, wp_v_ref[h, p], (((1,), (1,)), ((), ())),
                                 preferred_element_type=jnp.float32)
            ov = ov + bp_v_ref[h, p][None, :]
            acc_k = acc_k + s3[p] * ok
            acc_v = acc_v + s3[p] * ov
        acc_k_list.append(acc_k)
        acc_v_list.append(acc_v)
    nk_flat = jnp.concatenate(acc_k_list, axis=1)
    nv_flat = jnp.concatenate(acc_v_list, axis=1)
    out = jnp.concatenate(
        [jnp.broadcast_to(nk_flat[None, :, :], (_LEN, _B, _D)),
         jnp.broadcast_to(nv_flat[None, :, :], (_LEN, _B, _D))], axis=0)
    out_ref[...] = out


_NVREG = _LPAD // 16
_JW = _D // 128
_DPAD = 104
_DEPOFF = _DPAD + 8
_COMB = _DEPOFF + _DPAD


def _sc_route_body(comb_hbm, out_hbm, bp_hbm, sim_v, rows_v, out_v, bp_v,
                   sem):
    wid = lax.axis_index("s") * 2 + lax.axis_index("c")

    @pl.when(wid < _JW)
    def _():
        j = wid
        pltpu.sync_copy(comb_hbm.at[pl.ds(_DPAD, 8), pl.ds(0, 128)], sim_v)
        lanes = lax.iota(jnp.int32, 16)
        ids = [lanes + i * 16 for i in range(_NVREG)]
        tops = []
        for b in range(_B):
            vs = [sim_v[b, pl.ds(i * 16, 16)] for i in range(_NVREG)]
            for _ in range(_TKL):
                mv, mi = vs[0], ids[0]
                for i in range(1, _NVREG):
                    upd = vs[i] > mv
                    mv = jnp.where(upd, vs[i], mv)
                    mi = jnp.where(upd, ids[i], mi)
                mval = jnp.max(mv)
                ridx = jnp.min(jnp.where(mv == mval, mi, _BIGI))
                tops.append((mval, ridx // 8 * 8, ridx % 8))
                vs = [jnp.where(ids[i] == ridx, _NEG, vs[i])
                      for i in range(_NVREG)]
        cps = []
        for k, (_, rbase, _) in enumerate(tops):
            rb = pl.multiple_of(rbase, 8)
            cps.append(pltpu.async_copy(
                comb_hbm.at[pl.ds(rb, 8), pl.ds(j * 128, 128)],
                rows_v.at[k], sem))
            rb2 = pl.multiple_of(rbase + _DEPOFF, 8)
            cps.append(pltpu.async_copy(
                comb_hbm.at[pl.ds(rb2, 8), pl.ds(j * 128, 128)],
                rows_v.at[_B * _TKL + k], sem))
        for cp in cps:
            cp.wait()
        for b in range(_B):
            accs = [jnp.zeros((16,), jnp.float32) for _ in range(8)]
            accs2 = [jnp.zeros((16,), jnp.float32) for _ in range(8)]
            for t in range(_TKL):
                k = b * _TKL + t
                mval, _, rr = tops[k]
                for c in range(8):
                    accs[c] = accs[c] + mval * rows_v[k, rr, pl.ds(c * 16, 16)]
                    accs2[c] = accs2[c] + mval * rows_v[_B * _TKL + k, rr,
                                                        pl.ds(c * 16, 16)]
            for c in range(8):
                out_v[b, pl.ds(c * 16, 16)] = accs[c]
                bp_v[b, pl.ds(c * 16, 16)] = accs2[c]
        pltpu.sync_copy(out_v, out_hbm.at[:, pl.ds(j * 128, 128)])
        pltpu.sync_copy(bp_v, bp_hbm.at[:, pl.ds(j * 128, 128)])


_sc_route = pl.kernel(
    _sc_route_body,
    out_type=[jax.ShapeDtypeStruct((_B, _D), jnp.float32),
              jax.ShapeDtypeStruct((_B, _D), jnp.float32)],
    mesh=plsc.VectorSubcoreMesh(core_axis_name="c", subcore_axis_name="s",
                                num_cores=2, num_subcores=16),
    compiler_params=pltpu.CompilerParams(needs_layout_passes=False,
                                         skip_device_barrier=True),
    scratch_types=[
        pltpu.VMEM((8, 128), jnp.float32),
        pltpu.VMEM((2 * _B * _TKL, 8, 128), jnp.float32),
        pltpu.VMEM((_B, 128), jnp.float32),
        pltpu.VMEM((_B, 128), jnp.float32),
        pltpu.SemaphoreType.DMA,
    ],
)


def kernel(x_embed, prompt_key, new_desc_embed, w_text, w_prompt_proj,
           w_qkv_k, w_proj_k, b_proj_k, w_qkv_v, w_proj_v, b_proj_v):
    similarity, comb = pl.pallas_call(
        _tc_a_body,
        grid=(_GS,),
        in_specs=[
            pl.BlockSpec((_B, _SCHUNK, _D), lambda g: (0, g, 0)),
            pl.BlockSpec((_LPAD, _D), lambda g: (0, 0)),
            pl.BlockSpec((_LMAX, _D), lambda g: (0, 0)),
            pl.BlockSpec((_D, _D), lambda g: (0, 0)),
            pl.BlockSpec((_D, _D), lambda g: (0, 0)),
        ],
        out_specs=[
            pl.BlockSpec((_B, _LMAX), lambda g: (0, 0)),
            pl.BlockSpec((_COMB, _D), lambda g: (0, 0)),
        ],
        out_shape=[
            jax.ShapeDtypeStruct((_B, _LMAX), jnp.float32),
            jax.ShapeDtypeStruct((_COMB, _D), jnp.float32),
        ],
        scratch_shapes=[pltpu.VMEM((_B, _D), jnp.float32)],
    )(x_embed, prompt_key, new_desc_embed, w_text, w_prompt_proj)

    desc_out, bp = _sc_route(comb)

    out_bp = pl.pallas_call(
        _tc_c_body,
        grid=(1,),
        in_specs=[
            pl.BlockSpec((_B, _D), lambda i: (0, 0)),
            pl.BlockSpec((_B, _LMAX), lambda i: (0, 0)),
            pl.BlockSpec((_H, _TKL, _HD, _HD), lambda i: (0, 0, 0, 0)),
            pl.BlockSpec((_H, _TKL, _HD, _HD), lambda i: (0, 0, 0, 0)),
            pl.BlockSpec((_H, _TKL, _HD), lambda i: (0, 0, 0)),
            pl.BlockSpec((_H, _TKL, _HD, _HD), lambda i: (0, 0, 0, 0)),
            pl.BlockSpec((_H, _TKL, _HD, _HD), lambda i: (0, 0, 0, 0)),
            pl.BlockSpec((_H, _TKL, _HD), lambda i: (0, 0, 0)),
        ],
        out_specs=pl.BlockSpec((2 * _LEN, _B, _D), lambda i: (0, 0, 0)),
        out_shape=jax.ShapeDtypeStruct((2 * _LEN, _B, _D), jnp.float32),
    )(bp, similarity, lax.slice_in_dim(w_qkv_k, 2 * _HD, 3 * _HD, axis=2),
      w_proj_k, b_proj_k, lax.slice_in_dim(w_qkv_v, 2 * _HD, 3 * _HD, axis=2),
      w_proj_v, b_proj_v)

    return (similarity, desc_out, jnp.transpose(out_bp, (1, 0, 2)))
